# Initial kernel scaffold; baseline (speedup 1.0000x reference)
import jax
import jax.numpy as jnp
from jax.experimental import pallas as pl


def kernel(x_t, x_t_batch, smile_latent, W_U1, b_U1, W_U2, b_U2, W_W1, b_W1, W_W2, b_W2):
    raise NotImplementedError("write your pallas kernel here")


# speedup vs baseline: 2.1809x; 2.1809x over previous
import jax
import jax.numpy as jnp
from jax import lax
from jax.experimental import pallas as pl
from jax.experimental.pallas import tpu as pltpu

_N = 32768
_B = 16
_BN = 2048
_NB = _N // _BN
_NEG = -1.0e30


def _scores_body(x_ref, ids_ref, sl_ref, wu1_ref, bu1_ref, wu2_ref, bu2_ref,
                 ww1_ref, bw1_ref, w2r_ref, bw2_ref, e_ref, st_ref,
                 wx_ref, c_ref):
  i = pl.program_id(0)

  @pl.when(i == 0)
  def _init():
    ww1 = ww1_ref[...]
    w1_top = ww1[:150, :]
    w1_bot = ww1[150:, :]
    wx_ref[...] = jnp.dot(wu2_ref[...], w1_bot,
                          preferred_element_type=jnp.float32)
    u1 = jnp.dot(sl_ref[...], wu1_ref[...],
                 preferred_element_type=jnp.float32) + bu1_ref[...]
    c_ref[...] = (jnp.dot(u1, w1_top, preferred_element_type=jnp.float32)
                  + jnp.dot(bu2_ref[...], w1_bot,
                            preferred_element_type=jnp.float32)
                  + bw1_ref[...])
    st_ref[0:1, :] = jnp.full((1, _B), _NEG, jnp.float32)
    st_ref[1:2, :] = jnp.zeros((1, _B), jnp.float32)

  ids = ids_ref[...]
  oh = (ids == lax.broadcasted_iota(jnp.int32, (1, _B), 1)
        ).astype(jnp.float32)

  pre = jnp.dot(x_ref[...], wx_ref[...], preferred_element_type=jnp.float32)
  cg = jnp.dot(oh, c_ref[...], preferred_element_type=jnp.float32)
  h = jnp.tanh(pre + cg)
  e = jnp.sum(h * w2r_ref[...], axis=1, keepdims=True) + bw2_ref[...]
  e_ref[...] = e

  m_old = st_ref[0:1, :]
  m_part = jnp.max(jnp.where(oh > 0.5, e, _NEG), axis=0, keepdims=True)
  m_new = jnp.maximum(m_old, m_part)
  m_g = jnp.sum(oh * m_new, axis=1, keepdims=True)
  ex = jnp.exp(e - m_g)
  s_part = jnp.sum(oh * ex, axis=0, keepdims=True)
  st_ref[1:2, :] = st_ref[1:2, :] * jnp.exp(m_old - m_new) + s_part
  st_ref[0:1, :] = m_new


def _norm_body(e_ref, ids_ref, st_ref, a_ref):
  ids = ids_ref[...]
  oh = (ids == lax.broadcasted_iota(jnp.int32, (1, _B), 1)
        ).astype(jnp.float32)
  m_g = jnp.sum(oh * st_ref[0:1, :], axis=1, keepdims=True)
  s_g = jnp.sum(oh * st_ref[1:2, :], axis=1, keepdims=True)
  a_ref[...] = jnp.exp(e_ref[...] - m_g) / (s_g + 1e-16)


def kernel(x_t, x_t_batch, smile_latent, W_U1, b_U1, W_U2, b_U2,
           W_W1, b_W1, W_W2, b_W2):
  ids = x_t_batch.astype(jnp.int32).reshape(_N, 1)
  e, st = pl.pallas_call(
      _scores_body,
      grid=(_NB,),
      in_specs=[
          pl.BlockSpec((_BN, 93), lambda i: (i, 0)),
          pl.BlockSpec((_BN, 1), lambda i: (i, 0)),
          pl.BlockSpec((16, 500), lambda i: (0, 0)),
          pl.BlockSpec((500, 150), lambda i: (0, 0)),
          pl.BlockSpec((1, 150), lambda i: (0, 0)),
          pl.BlockSpec((93, 150), lambda i: (0, 0)),
          pl.BlockSpec((1, 150), lambda i: (0, 0)),
          pl.BlockSpec((300, 150), lambda i: (0, 0)),
          pl.BlockSpec((1, 150), lambda i: (0, 0)),
          pl.BlockSpec((1, 150), lambda i: (0, 0)),
          pl.BlockSpec((1, 1), lambda i: (0, 0)),
      ],
      out_specs=[
          pl.BlockSpec((_BN, 1), lambda i: (i, 0)),
          pl.BlockSpec((2, _B), lambda i: (0, 0)),
      ],
      out_shape=[
          jax.ShapeDtypeStruct((_N, 1), jnp.float32),
          jax.ShapeDtypeStruct((2, _B), jnp.float32),
      ],
      scratch_shapes=[
          pltpu.VMEM((93, 150), jnp.float32),
          pltpu.VMEM((16, 150), jnp.float32),
      ],
  )(x_t, ids, smile_latent, W_U1, b_U1.reshape(1, 150), W_U2,
    b_U2.reshape(1, 150), W_W1, b_W1.reshape(1, 150), W_W2.reshape(1, 150),
    b_W2.reshape(1, 1))

  alpha = pl.pallas_call(
      _norm_body,
      grid=(_NB,),
      in_specs=[
          pl.BlockSpec((_BN, 1), lambda i: (i, 0)),
          pl.BlockSpec((_BN, 1), lambda i: (i, 0)),
          pl.BlockSpec((2, _B), lambda i: (0, 0)),
      ],
      out_specs=pl.BlockSpec((_BN, 1), lambda i: (i, 0)),
      out_shape=jax.ShapeDtypeStruct((_N, 1), jnp.float32),
  )(e, ids, st)
  return alpha


# speedup vs baseline: 5.3737x; 2.4639x over previous
import jax
import jax.numpy as jnp
from jax import lax
from jax.experimental import pallas as pl
from jax.experimental.pallas import tpu as pltpu

_N = 32768
_B = 16
_BN = 2048
_NB = _N // _BN
_NEG = -1.0e30


def _dotg(a, b, dims):
  return lax.dot_general(a, b, (dims, ((), ())),
                         preferred_element_type=jnp.float32)


def _scores_body(x_ref, ids_ref, sl_ref, wu1_ref, bu1_ref, wu2_ref, bu2_ref,
                 ww1_ref, bw1_ref, ww2_ref, bw2_ref,
                 e_ref, m_ref, s_ref, wxt_ref, ct_ref):
  i = pl.program_id(0)

  @pl.when(i == 0)
  def _init():
    ww1 = ww1_ref[...]
    w1_top = ww1[:150, :]
    w1_bot = ww1[150:, :]
    wxt_ref[...] = _dotg(w1_bot, wu2_ref[...], ((0,), (1,)))
    u1 = jnp.dot(sl_ref[...], wu1_ref[...],
                 preferred_element_type=jnp.float32) + bu1_ref[...]
    ct_ref[...] = (_dotg(w1_top, u1, ((0,), (1,)))
                   + _dotg(w1_bot, bu2_ref[...], ((0,), (1,)))
                   + bw1_ref[...])
    m_ref[...] = jnp.full((_B, 1), _NEG, jnp.float32)
    s_ref[...] = jnp.zeros((_B, 1), jnp.float32)

  ids = ids_ref[...]
  ohb = ids == lax.broadcasted_iota(jnp.int32, (_B, 1), 0)

  pre = _dotg(wxt_ref[...], x_ref[...], ((1,), (1,)))
  cg = _dotg(ct_ref[...], ohb.astype(jnp.float32), ((1,), (0,)))
  ht = jnp.tanh(pre + cg)
  et = _dotg(ww2_ref[...], ht, ((0,), (0,))) + bw2_ref[...]
  e_ref[...] = et

  m_part = jnp.max(jnp.where(ohb, et, _NEG), axis=1, keepdims=True)
  s_part = jnp.sum(jnp.where(ohb, jnp.exp(et), 0.0), axis=1, keepdims=True)
  m_ref[...] = jnp.maximum(m_ref[...], m_part)
  s_ref[...] = s_ref[...] + s_part


def _norm_body(e_ref, ids_ref, m_ref, s_ref, a_ref):
  ids = ids_ref[...]
  ohf = (ids == lax.broadcasted_iota(jnp.int32, (_B, 1), 0)
         ).astype(jnp.float32)
  m = jnp.maximum(m_ref[...], -80.0)
  em = jnp.exp(-m)
  s = s_ref[...]
  q = jnp.where(s > 0.0, em / (em * s + 1e-16), 0.0)
  qg = _dotg(q, ohf, ((0,), (0,)))
  a_ref[...] = jnp.exp(e_ref[...]) * qg


def kernel(x_t, x_t_batch, smile_latent, W_U1, b_U1, W_U2, b_U2,
           W_W1, b_W1, W_W2, b_W2):
  ids = x_t_batch.astype(jnp.int32).reshape(1, _N)
  e, m, s = pl.pallas_call(
      _scores_body,
      grid=(_NB,),
      in_specs=[
          pl.BlockSpec((_BN, 93), lambda i: (i, 0)),
          pl.BlockSpec((1, _BN), lambda i: (0, i)),
          pl.BlockSpec((16, 500), lambda i: (0, 0)),
          pl.BlockSpec((500, 150), lambda i: (0, 0)),
          pl.BlockSpec((1, 150), lambda i: (0, 0)),
          pl.BlockSpec((93, 150), lambda i: (0, 0)),
          pl.BlockSpec((1, 150), lambda i: (0, 0)),
          pl.BlockSpec((300, 150), lambda i: (0, 0)),
          pl.BlockSpec((150, 1), lambda i: (0, 0)),
          pl.BlockSpec((150, 1), lambda i: (0, 0)),
          pl.BlockSpec((1, 1), lambda i: (0, 0)),
      ],
      out_specs=[
          pl.BlockSpec((1, _BN), lambda i: (0, i)),
          pl.BlockSpec((_B, 1), lambda i: (0, 0)),
          pl.BlockSpec((_B, 1), lambda i: (0, 0)),
      ],
      out_shape=[
          jax.ShapeDtypeStruct((1, _N), jnp.float32),
          jax.ShapeDtypeStruct((_B, 1), jnp.float32),
          jax.ShapeDtypeStruct((_B, 1), jnp.float32),
      ],
      scratch_shapes=[
          pltpu.VMEM((150, 93), jnp.float32),
          pltpu.VMEM((150, _B), jnp.float32),
      ],
  )(x_t, ids, smile_latent, W_U1, b_U1.reshape(1, 150), W_U2,
    b_U2.reshape(1, 150), W_W1, b_W1.reshape(150, 1), W_W2,
    b_W2.reshape(1, 1))

  alpha = pl.pallas_call(
      _norm_body,
      grid=(_NB,),
      in_specs=[
          pl.BlockSpec((1, _BN), lambda i: (0, i)),
          pl.BlockSpec((1, _BN), lambda i: (0, i)),
          pl.BlockSpec((_B, 1), lambda i: (0, 0)),
          pl.BlockSpec((_B, 1), lambda i: (0, 0)),
      ],
      out_specs=pl.BlockSpec((1, _BN), lambda i: (0, i)),
      out_shape=jax.ShapeDtypeStruct((1, _N), jnp.float32),
  )(e, ids, m, s)
  return alpha.reshape(_N, 1)


# speedup vs baseline: 7.3548x; 1.3687x over previous
import jax
import jax.numpy as jnp
from jax import lax
from jax.experimental import pallas as pl
from jax.experimental.pallas import tpu as pltpu

_N = 32768
_B = 16
_BN = 4096
_NB = _N // _BN
_NEG = -1.0e30


def _dotg(a, b, dims):
  return lax.dot_general(a, b, (dims, ((), ())),
                         preferred_element_type=jnp.float32)


def _body(x_ref, ids_ref, idsf_ref, sl_ref, wu1_ref, bu1_ref, wu2_ref,
          bu2_ref, ww1_ref, bw1_ref, ww2_ref, bw2_ref,
          a_ref, wxt_ref, ct_ref, m_ref, s_ref, e_ref):
  i = pl.program_id(0)

  @pl.when(i == 0)
  def _init():
    ww1 = ww1_ref[...]
    w1_top = ww1[:150, :]
    w1_bot = ww1[150:, :]
    wxt_ref[...] = _dotg(w1_bot, wu2_ref[...], ((0,), (1,)))
    u1 = jnp.dot(sl_ref[...], wu1_ref[...],
                 preferred_element_type=jnp.float32) + bu1_ref[...]
    ct_ref[...] = (_dotg(w1_top, u1, ((0,), (1,)))
                   + _dotg(w1_bot, bu2_ref[...], ((0,), (1,)))
                   + bw1_ref[...])
    m_ref[...] = jnp.full((_B, 1), _NEG, jnp.float32)
    s_ref[...] = jnp.zeros((_B, 1), jnp.float32)

  @pl.when(i < _NB)
  def _scores():
    ids = ids_ref[...]
    ohb = ids == lax.broadcasted_iota(jnp.int32, (_B, 1), 0)
    pre = _dotg(wxt_ref[...], x_ref[...], ((1,), (1,)))
    cg = _dotg(ct_ref[...], ohb.astype(jnp.float32), ((1,), (0,)))
    ht = jnp.tanh(pre + cg)
    et = _dotg(ww2_ref[...], ht, ((0,), (0,))) + bw2_ref[...]
    e_ref[pl.ds(i, 1), :] = et
    m_part = jnp.max(jnp.where(ohb, et, _NEG), axis=1, keepdims=True)
    s_part = jnp.sum(jnp.where(ohb, jnp.exp(et), 0.0), axis=1, keepdims=True)
    m_ref[...] = jnp.maximum(m_ref[...], m_part)
    s_ref[...] = s_ref[...] + s_part

  @pl.when(i == _NB)
  def _normalize():
    m = jnp.maximum(m_ref[...], -80.0)
    em = jnp.exp(-m)
    s = s_ref[...]
    q = jnp.where(s > 0.0, em / (em * s + 1e-16), 0.0)
    iota_b = lax.broadcasted_iota(jnp.int32, (_B, 1), 0)
    for j in range(_NB):
      ids_j = idsf_ref[0:1, j * _BN:(j + 1) * _BN]
      ohf = (ids_j == iota_b).astype(jnp.float32)
      qg = _dotg(q, ohf, ((0,), (0,)))
      a_ref[0:1, j * _BN:(j + 1) * _BN] = (
          jnp.exp(e_ref[j:j + 1, :]) * qg)


def kernel(x_t, x_t_batch, smile_latent, W_U1, b_U1, W_U2, b_U2,
           W_W1, b_W1, W_W2, b_W2):
  ids = x_t_batch.astype(jnp.int32).reshape(1, _N)
  last = _NB - 1
  alpha = pl.pallas_call(
      _body,
      grid=(_NB + 1,),
      in_specs=[
          pl.BlockSpec((_BN, 93), lambda i: (jnp.minimum(i, last), 0)),
          pl.BlockSpec((1, _BN), lambda i: (0, jnp.minimum(i, last))),
          pl.BlockSpec((1, _N), lambda i: (0, 0)),
          pl.BlockSpec((16, 500), lambda i: (0, 0)),
          pl.BlockSpec((500, 150), lambda i: (0, 0)),
          pl.BlockSpec((1, 150), lambda i: (0, 0)),
          pl.BlockSpec((93, 150), lambda i: (0, 0)),
          pl.BlockSpec((1, 150), lambda i: (0, 0)),
          pl.BlockSpec((300, 150), lambda i: (0, 0)),
          pl.BlockSpec((150, 1), lambda i: (0, 0)),
          pl.BlockSpec((150, 1), lambda i: (0, 0)),
          pl.BlockSpec((1, 1), lambda i: (0, 0)),
      ],
      out_specs=pl.BlockSpec((1, _N), lambda i: (0, 0)),
      out_shape=jax.ShapeDtypeStruct((1, _N), jnp.float32),
      scratch_shapes=[
          pltpu.VMEM((150, 93), jnp.float32),
          pltpu.VMEM((150, _B), jnp.float32),
          pltpu.VMEM((_B, 1), jnp.float32),
          pltpu.VMEM((_B, 1), jnp.float32),
          pltpu.VMEM((_NB, _BN), jnp.float32),
      ],
  )(x_t, ids, ids, smile_latent, W_U1, b_U1.reshape(1, 150), W_U2,
    b_U2.reshape(1, 150), W_W1, b_W1.reshape(150, 1), W_W2,
    b_W2.reshape(1, 1))
  return alpha.reshape(_N, 1)


# speedup vs baseline: 7.6567x; 1.0410x over previous
import jax
import jax.numpy as jnp
from jax import lax
from jax.experimental import pallas as pl
from jax.experimental.pallas import tpu as pltpu

_N = 32768
_B = 16
_BN = 8192
_NB = _N // _BN
_NEG = -1.0e30


def _dotg(a, b, dims):
  return lax.dot_general(a, b, (dims, ((), ())),
                         preferred_element_type=jnp.float32)


def _body(x_ref, ids_ref, idsf_ref, sl_ref, wu1_ref, bu1_ref, wu2_ref,
          bu2_ref, ww1_ref, bw1_ref, ww2_ref, bw2_ref,
          a_ref, wxt_ref, ct_ref, m_ref, s_ref, e_ref):
  i = pl.program_id(0)

  @pl.when(i == 0)
  def _init():
    ww1 = ww1_ref[...]
    w1_top = ww1[:150, :]
    w1_bot = ww1[150:, :]
    wxt_ref[...] = _dotg(w1_bot, wu2_ref[...], ((0,), (1,)))
    u1 = jnp.dot(sl_ref[...], wu1_ref[...],
                 preferred_element_type=jnp.float32) + bu1_ref[...]
    ct_ref[...] = (_dotg(w1_top, u1, ((0,), (1,)))
                   + _dotg(w1_bot, bu2_ref[...], ((0,), (1,)))
                   + bw1_ref[...])
    m_ref[...] = jnp.full((_B, 1), _NEG, jnp.float32)
    s_ref[...] = jnp.zeros((_B, 1), jnp.float32)

  @pl.when(i < _NB)
  def _scores():
    ids = ids_ref[...]
    ohb = ids == lax.broadcasted_iota(jnp.int32, (_B, 1), 0)
    pre = _dotg(wxt_ref[...], x_ref[...], ((1,), (1,)))
    cg = _dotg(ct_ref[...], ohb.astype(jnp.float32), ((1,), (0,)))
    ht = jnp.tanh(pre + cg)
    et = _dotg(ww2_ref[...], ht, ((0,), (0,))) + bw2_ref[...]
    e_ref[pl.ds(i, 1), :] = et
    m_part = jnp.max(jnp.where(ohb, et, _NEG), axis=1, keepdims=True)
    s_part = jnp.sum(jnp.where(ohb, jnp.exp(et), 0.0), axis=1, keepdims=True)
    m_ref[...] = jnp.maximum(m_ref[...], m_part)
    s_ref[...] = s_ref[...] + s_part

  @pl.when(i == _NB)
  def _normalize():
    m = jnp.maximum(m_ref[...], -80.0)
    em = jnp.exp(-m)
    s = s_ref[...]
    q = jnp.where(s > 0.0, em / (em * s + 1e-16), 0.0)
    iota_b = lax.broadcasted_iota(jnp.int32, (_B, 1), 0)
    for j in range(_NB):
      ids_j = idsf_ref[0:1, j * _BN:(j + 1) * _BN]
      ohf = (ids_j == iota_b).astype(jnp.float32)
      qg = _dotg(q, ohf, ((0,), (0,)))
      a_ref[0:1, j * _BN:(j + 1) * _BN] = (
          jnp.exp(e_ref[j:j + 1, :]) * qg)


def kernel(x_t, x_t_batch, smile_latent, W_U1, b_U1, W_U2, b_U2,
           W_W1, b_W1, W_W2, b_W2):
  ids = x_t_batch.astype(jnp.int32).reshape(1, _N)
  last = _NB - 1
  alpha = pl.pallas_call(
      _body,
      grid=(_NB + 1,),
      in_specs=[
          pl.BlockSpec((_BN, 93), lambda i: (jnp.minimum(i, last), 0)),
          pl.BlockSpec((1, _BN), lambda i: (0, jnp.minimum(i, last))),
          pl.BlockSpec((1, _N), lambda i: (0, 0)),
          pl.BlockSpec((16, 500), lambda i: (0, 0)),
          pl.BlockSpec((500, 150), lambda i: (0, 0)),
          pl.BlockSpec((1, 150), lambda i: (0, 0)),
          pl.BlockSpec((93, 150), lambda i: (0, 0)),
          pl.BlockSpec((1, 150), lambda i: (0, 0)),
          pl.BlockSpec((300, 150), lambda i: (0, 0)),
          pl.BlockSpec((150, 1), lambda i: (0, 0)),
          pl.BlockSpec((150, 1), lambda i: (0, 0)),
          pl.BlockSpec((1, 1), lambda i: (0, 0)),
      ],
      out_specs=pl.BlockSpec((1, _N), lambda i: (0, 0)),
      out_shape=jax.ShapeDtypeStruct((1, _N), jnp.float32),
      scratch_shapes=[
          pltpu.VMEM((150, 93), jnp.float32),
          pltpu.VMEM((150, _B), jnp.float32),
          pltpu.VMEM((_B, 1), jnp.float32),
          pltpu.VMEM((_B, 1), jnp.float32),
          pltpu.VMEM((_NB, _BN), jnp.float32),
      ],
  )(x_t, ids, ids, smile_latent, W_U1, b_U1.reshape(1, 150), W_U2,
    b_U2.reshape(1, 150), W_W1, b_W1.reshape(150, 1), W_W2,
    b_W2.reshape(1, 1))
  return alpha.reshape(_N, 1)
